# Initial kernel scaffold; baseline (speedup 1.0000x reference)
import jax
import jax.numpy as jnp
from jax.experimental import pallas as pl


def kernel(values, arousal_bins, dominance_bins, valence_bins):
    raise NotImplementedError("write your pallas kernel here")


# speedup vs baseline: 51.3143x; 51.3143x over previous
import functools

import jax
import jax.numpy as jnp
from jax import lax
from jax.experimental import pallas as pl
from jax.experimental.pallas import tpu as pltpu
from jax.experimental.pallas import tpu_sc as plsc

_N = 1000000
_FLAT = 3 * _N
_NC = 2
_NS = 16
_NW = _NC * _NS
_LANES = 16
_PER_TILE = 93888
_PAD_FLAT = _NW * _PER_TILE
_NCHUNK = 3
_CHUNK = _PER_TILE // _NCHUNK
_GROUP = 6
_NGROUP = _CHUNK // (_LANES * _GROUP)
_STEPS = [128, 64, 32, 16, 8, 4, 2, 1]


def _qbody(vals_hbm, table_hbm, out_hbm, table_v, in_v, out_v):
    wid = lax.axis_index("s") * _NC + lax.axis_index("c")
    base = wid * _PER_TILE
    pltpu.sync_copy(table_hbm, table_v)
    iota = lax.iota(jnp.int32, _LANES)
    i0 = [(lax.rem(iota + u, 3) << 8) + 127 for u in range(3)]

    for c in range(_NCHUNK):
        start = base + c * _CHUNK
        pltpu.sync_copy(vals_hbm.at[pl.ds(start, _CHUNK)], in_v)

        @plsc.parallel_loop(0, _NGROUP, 1, unroll=2)
        def vbody(g):
            goff = g * (_LANES * _GROUP)
            for u in range(_GROUP):
                off = goff + u * _LANES
                x = in_v[pl.ds(off, _LANES)]
                i = i0[u % 3]
                for k, s in enumerate(_STEPS):
                    b = plsc.load_gather(table_v, [i])
                    m = b <= x
                    s_next = _STEPS[k + 1] if k + 1 < len(_STEPS) else 1
                    i = i + jnp.where(m, s_next, s_next - s)
                out_v[pl.ds(off, _LANES)] = i & 255
        pltpu.sync_copy(out_v, out_hbm.at[pl.ds(start, _CHUNK)])


def kernel(values, arousal_bins, dominance_bins, valence_bins):
    flat = jnp.pad(jnp.reshape(values, (-1,)), (0, _PAD_FLAT - _FLAT))
    table = jnp.concatenate([arousal_bins, dominance_bins, valence_bins])
    run = pl.kernel(
        _qbody,
        out_type=jax.ShapeDtypeStruct((_PAD_FLAT,), jnp.int32),
        mesh=plsc.VectorSubcoreMesh(core_axis_name="c", subcore_axis_name="s"),
        compiler_params=pltpu.CompilerParams(needs_layout_passes=False),
        scratch_types=[
            pltpu.VMEM((3 * 256,), jnp.float32),
            pltpu.VMEM((_CHUNK,), jnp.float32),
            pltpu.VMEM((_CHUNK,), jnp.int32),
        ],
    )
    out = run(flat, table)
    return out[:_FLAT].reshape(_N, 3)
